# baseline (speedup 1.0000x reference)
import jax
import jax.numpy as jnp
from jax.experimental import pallas as pl
from jax.experimental.pallas import tpu as pltpu

_ROWS = 100000
_COLS = 512
_NCHUNKS = 10
_CHUNK = _ROWS // _NCHUNKS


def _body(x_hbm, o_hbm, head_vmem, sems, head_sem):
    head_in = pltpu.make_async_copy(
        x_hbm.at[pl.ds(0, 8), :], head_vmem, head_sem)
    head_in.start()
    copies = []
    for i in range(_NCHUNKS):
        c = pltpu.make_async_copy(
            x_hbm.at[pl.ds(i * _CHUNK, _CHUNK), :],
            o_hbm.at[pl.ds(i * _CHUNK, _CHUNK), :],
            sems.at[i],
        )
        c.start()
        copies.append(c)
    head_in.wait()
    two = jnp.full((1, _COLS), 2.0, jnp.float32)
    head_vmem[pl.ds(1, 1), :] = two
    head_vmem[pl.ds(3, 1), :] = two
    for c in copies:
        c.wait()
    head_out = pltpu.make_async_copy(
        head_vmem, o_hbm.at[pl.ds(0, 8), :], head_sem)
    head_out.start()
    head_out.wait()


def kernel(x):
    return pl.pallas_call(
        _body,
        in_specs=[pl.BlockSpec(memory_space=pl.ANY)],
        out_specs=pl.BlockSpec(memory_space=pl.ANY),
        out_shape=jax.ShapeDtypeStruct((_ROWS, _COLS), jnp.float32),
        scratch_shapes=[
            pltpu.VMEM((8, _COLS), jnp.float32),
            pltpu.SemaphoreType.DMA((_NCHUNKS,)),
            pltpu.SemaphoreType.DMA,
        ],
    )(x)


# speedup vs baseline: 42.7625x; 42.7625x over previous
import jax
import jax.numpy as jnp
from jax import lax
from jax.experimental import pallas as pl
from jax.experimental.pallas import tpu as pltpu
from jax.experimental.pallas import tpu_sc as plsc

_ROWS = 100000
_COLS = 512
_BLOCK = 2000


def _copy_body(x_ref, o_ref):
    o_ref[...] = x_ref[...]


def _tc_copy(x):
    return pl.pallas_call(
        _copy_body,
        grid=(_ROWS // _BLOCK,),
        in_specs=[pl.BlockSpec((_BLOCK, _COLS), lambda i: (i, 0))],
        out_specs=pl.BlockSpec((_BLOCK, _COLS), lambda i: (i, 0)),
        out_shape=jax.ShapeDtypeStruct((_ROWS, _COLS), jnp.float32),
    )(x)


_mesh = plsc.VectorSubcoreMesh(core_axis_name="c", subcore_axis_name="s")


@pl.kernel(mesh=_mesh, scratch_types=[pltpu.VMEM((_COLS,), jnp.float32)])
def _sc_patch(out_ref, row_vmem):
    c = lax.axis_index("c")
    s = lax.axis_index("s")

    @pl.when(jnp.logical_and(c == 0, s == 0))
    def _():
        for i in range(_COLS // 16):
            row_vmem[pl.ds(i * 16, 16)] = jnp.full((16,), 2.0, jnp.float32)
        pltpu.sync_copy(row_vmem, out_ref.at[1])
        pltpu.sync_copy(row_vmem, out_ref.at[3])


def kernel(x):
    out_ref = jax.new_ref(_tc_copy(x))
    _sc_patch(out_ref)
    return jax.freeze(out_ref)
